# baseline (speedup 1.0000x reference)
import functools

import jax
import jax.numpy as jnp
from jax import lax
from jax.experimental import pallas as pl
from jax.experimental.pallas import tpu as pltpu
from jax.experimental.pallas import tpu_sc as plsc

BATCH = 16384
EMBED_DIM = 64
SHARED_DIM = 16
NUM_WORKERS = 32
B_PER_W = BATCH // NUM_WORKERS


def _sc_kernel():
    mesh = plsc.VectorSubcoreMesh(core_axis_name="c", subcore_axis_name="s")

    @functools.partial(
        pl.kernel,
        out_type=jax.ShapeDtypeStruct((BATCH, EMBED_DIM), jnp.float32),
        mesh=mesh,
        scratch_types=[
            pltpu.VMEM((B_PER_W,), jnp.int32),
            pltpu.VMEM((B_PER_W, EMBED_DIM), jnp.float32),
            pltpu.VMEM((SHARED_DIM,), jnp.float32),
            pltpu.SemaphoreType.DMA,
        ],
        compiler_params=pltpu.CompilerParams(use_tc_tiling_on_sc=False),
    )
    def k(x_hbm, w_hbm, sh_hbm, out_hbm, idx_v, rows_v, sh_v, sem):
        wid = lax.axis_index("s") * 2 + lax.axis_index("c")
        base = wid * B_PER_W
        pltpu.sync_copy(sh_hbm.at[0], sh_v)
        pltpu.sync_copy(x_hbm.at[pl.ds(base, B_PER_W)], idx_v)
        pltpu.async_copy(w_hbm.at[idx_v], rows_v, sem).wait()
        sh = sh_v[...]

        def body(r, carry):
            rows_v[r, pl.ds(0, SHARED_DIM)] = sh
            return carry

        lax.fori_loop(0, B_PER_W, body, 0, unroll=8)
        pltpu.sync_copy(rows_v, out_hbm.at[pl.ds(base, B_PER_W)])

    return k


_k = _sc_kernel()


def kernel(X, W, shared_embed):
    return _k(X.astype(jnp.int32), W, shared_embed)
